# Initial kernel scaffold; baseline (speedup 1.0000x reference)
import jax
import jax.numpy as jnp
from jax.experimental import pallas as pl


def kernel(x_0, x_1, x_2, x_3, x_4, adjacency_0, adjacency_1, adjacency_2, adjacency_3, adjacency_4, incidence_0_1, incidence_0_2, incidence_0_3, incidence_0_4, incidence_1_2, incidence_1_3, incidence_1_4, incidence_2_3, incidence_2_4, incidence_3_4, w_hbs0, w_hbs1, w_hbns_s, w_hbns_t):
    raise NotImplementedError("write your pallas kernel here")


# speedup vs baseline: 1.2004x; 1.2004x over previous
import jax
import jax.numpy as jnp
from jax.experimental import pallas as pl
from jax.experimental.pallas import tpu as pltpu

N0, N1 = 2048, 4096
D = 128
BI = 512
BJ = 512
A0K = N0 // 8
A1K = N1 // 4

_BF = jnp.bfloat16
_F32 = jnp.float32


def _gnn_kernel(x0_ref, x1_ref, a0_ref, a1_ref, b_ref,
                w0_ref, w1_ref, ws_ref, wt_ref,
                out0_ref, out1_ref,
                y0_ref, y1_ref, ys_ref, yt_ref):
    i = pl.program_id(0)
    j = pl.program_id(1)

    @pl.when((i == 0) & (j == 0))
    def _projections():
        x0b = x0_ref[...].astype(_BF)
        x1b = x1_ref[...].astype(_BF)
        y0_ref[...] = jnp.dot(x0b, w0_ref[...].astype(_BF),
                              preferred_element_type=_F32).astype(_BF)
        yt_ref[...] = jnp.dot(x0b, wt_ref[...].astype(_BF),
                              preferred_element_type=_F32).astype(_BF)
        y1_ref[...] = jnp.dot(x1b, w1_ref[...].astype(_BF),
                              preferred_element_type=_F32).astype(_BF)
        ys_ref[...] = jnp.dot(x1b, ws_ref[...].astype(_BF),
                              preferred_element_type=_F32).astype(_BF)

    a0 = a0_ref[...].astype(_BF)
    a1 = a1_ref[...].astype(_BF)
    b = b_ref[...].astype(_BF)

    m0 = jnp.dot(a0, y0_ref[pl.ds(j * A0K, A0K), :],
                 preferred_element_type=_F32)
    m0 += jnp.dot(b, ys_ref[pl.ds(j * BJ, BJ), :],
                  preferred_element_type=_F32)

    m1 = jnp.dot(a1, y1_ref[pl.ds(i * A1K, A1K), :],
                 preferred_element_type=_F32)
    m1 += jax.lax.dot_general(b, yt_ref[pl.ds(i * BI, BI), :],
                              dimension_numbers=(((0,), (0,)), ((), ())),
                              preferred_element_type=_F32)

    @pl.when(j == 0)
    def _():
        out0_ref[pl.ds(i * BI, BI), :] = m0

    @pl.when(j > 0)
    def _():
        out0_ref[pl.ds(i * BI, BI), :] += m0

    @pl.when(i == 0)
    def _():
        out1_ref[pl.ds(j * BJ, BJ), :] = m1

    @pl.when(i > 0)
    def _():
        out1_ref[pl.ds(j * BJ, BJ), :] += m1


def kernel(x_0, x_1, x_2, x_3, x_4, adjacency_0, adjacency_1, adjacency_2,
           adjacency_3, adjacency_4, incidence_0_1, incidence_0_2,
           incidence_0_3, incidence_0_4, incidence_1_2, incidence_1_3,
           incidence_1_4, incidence_2_3, incidence_2_4, incidence_3_4,
           w_hbs0, w_hbs1, w_hbns_s, w_hbns_t):
    grid = (N0 // BI, N1 // BJ)
    full = lambda shape: pl.BlockSpec(shape, lambda i, j: (0, 0))
    out0, out1 = pl.pallas_call(
        _gnn_kernel,
        grid=grid,
        in_specs=[
            full((N0, D)),
            full((N1, D)),
            pl.BlockSpec((BI, A0K), lambda i, j: (i, j)),
            pl.BlockSpec((BJ, A1K), lambda i, j: (j, i)),
            pl.BlockSpec((BI, BJ), lambda i, j: (i, j)),
            full((D, D)), full((D, D)), full((D, D)), full((D, D)),
        ],
        out_specs=[full((N0, D)), full((N1, D))],
        out_shape=[
            jax.ShapeDtypeStruct((N0, D), _F32),
            jax.ShapeDtypeStruct((N1, D), _F32),
        ],
        scratch_shapes=[
            pltpu.VMEM((N0, D), _BF),
            pltpu.VMEM((N1, D), _BF),
            pltpu.VMEM((N1, D), _BF),
            pltpu.VMEM((N0, D), _BF),
        ],
    )(x_0, x_1, adjacency_0, adjacency_1, incidence_0_1,
      w_hbs0, w_hbs1, w_hbns_s, w_hbns_t)
    return (out0, out1, x_2, x_3, x_4)


# speedup vs baseline: 1.4351x; 1.1955x over previous
import jax
import jax.numpy as jnp
from jax.experimental import pallas as pl
from jax.experimental.pallas import tpu as pltpu

N0, N1 = 2048, 4096
D = 128
BI = 512
BJ = 1024
GI = N0 // BI
GJ = N1 // BJ
A0K = N0 // GJ
A1K = N1 // GI

_BF = jnp.bfloat16
_F32 = jnp.float32


def _gnn_kernel(x0_ref, x1_ref, a0_ref, a1_ref, b_ref,
                w0_ref, w1_ref, ws_ref, wt_ref,
                out0_ref, out1_ref,
                y0_ref, y1_ref, ys_ref, yt_ref):
    i = pl.program_id(0)
    j = pl.program_id(1)

    @pl.when((i == 0) & (j == 0))
    def _projections():
        x0b = x0_ref[...].astype(_BF)
        x1b = x1_ref[...].astype(_BF)
        y0_ref[...] = jnp.dot(x0b, w0_ref[...].astype(_BF),
                              preferred_element_type=_F32).astype(_BF)
        yt_ref[...] = jnp.dot(x0b, wt_ref[...].astype(_BF),
                              preferred_element_type=_F32).astype(_BF)
        y1_ref[...] = jnp.dot(x1b, w1_ref[...].astype(_BF),
                              preferred_element_type=_F32).astype(_BF)
        ys_ref[...] = jnp.dot(x1b, ws_ref[...].astype(_BF),
                              preferred_element_type=_F32).astype(_BF)

    a0 = a0_ref[...].astype(_BF)
    a1 = a1_ref[...].astype(_BF)
    b = b_ref[...].astype(_BF)

    m0 = jnp.dot(a0, y0_ref[pl.ds(j * A0K, A0K), :],
                 preferred_element_type=_F32)
    m0 += jnp.dot(b, ys_ref[pl.ds(j * BJ, BJ), :],
                  preferred_element_type=_F32)

    m1 = jnp.dot(a1, y1_ref[pl.ds(i * A1K, A1K), :],
                 preferred_element_type=_F32)
    m1 += jax.lax.dot_general(b, yt_ref[pl.ds(i * BI, BI), :],
                              dimension_numbers=(((0,), (0,)), ((), ())),
                              preferred_element_type=_F32)

    @pl.when(j == 0)
    def _():
        out0_ref[pl.ds(i * BI, BI), :] = m0

    @pl.when(j > 0)
    def _():
        out0_ref[pl.ds(i * BI, BI), :] += m0

    @pl.when(i == 0)
    def _():
        out1_ref[pl.ds(j * BJ, BJ), :] = m1

    @pl.when(i > 0)
    def _():
        out1_ref[pl.ds(j * BJ, BJ), :] += m1


def kernel(x_0, x_1, x_2, x_3, x_4, adjacency_0, adjacency_1, adjacency_2,
           adjacency_3, adjacency_4, incidence_0_1, incidence_0_2,
           incidence_0_3, incidence_0_4, incidence_1_2, incidence_1_3,
           incidence_1_4, incidence_2_3, incidence_2_4, incidence_3_4,
           w_hbs0, w_hbs1, w_hbns_s, w_hbns_t):
    grid = (GI, GJ)
    full = lambda shape: pl.BlockSpec(shape, lambda i, j: (0, 0))
    out0, out1 = pl.pallas_call(
        _gnn_kernel,
        grid=grid,
        in_specs=[
            full((N0, D)),
            full((N1, D)),
            pl.BlockSpec((BI, A0K), lambda i, j: (i, j)),
            pl.BlockSpec((BJ, A1K), lambda i, j: (j, i)),
            pl.BlockSpec((BI, BJ), lambda i, j: (i, j)),
            full((D, D)), full((D, D)), full((D, D)), full((D, D)),
        ],
        out_specs=[full((N0, D)), full((N1, D))],
        out_shape=[
            jax.ShapeDtypeStruct((N0, D), _F32),
            jax.ShapeDtypeStruct((N1, D), _F32),
        ],
        scratch_shapes=[
            pltpu.VMEM((N0, D), _BF),
            pltpu.VMEM((N1, D), _BF),
            pltpu.VMEM((N1, D), _BF),
            pltpu.VMEM((N0, D), _BF),
        ],
    )(x_0, x_1, adjacency_0, adjacency_1, incidence_0_1,
      w_hbs0, w_hbs1, w_hbns_s, w_hbns_t)
    return (out0, out1, x_2, x_3, x_4)


# speedup vs baseline: 1.4610x; 1.0180x over previous
import jax
import jax.numpy as jnp
from jax.experimental import pallas as pl
from jax.experimental.pallas import tpu as pltpu

N0, N1 = 2048, 4096
D = 128
BI = 1024
BJ = 1024
GI = N0 // BI
GJ = N1 // BJ
A0K = N0 // GJ
A1K = N1 // GI

_BF = jnp.bfloat16
_F32 = jnp.float32


def _gnn_kernel(x0_ref, x1_ref, a0_ref, a1_ref, b_ref,
                w0_ref, w1_ref, ws_ref, wt_ref,
                out0_ref, out1_ref,
                y0_ref, y1_ref, ys_ref, yt_ref):
    i = pl.program_id(0)
    j = pl.program_id(1)

    @pl.when((i == 0) & (j == 0))
    def _projections():
        x0b = x0_ref[...].astype(_BF)
        x1b = x1_ref[...].astype(_BF)
        y0_ref[...] = jnp.dot(x0b, w0_ref[...].astype(_BF),
                              preferred_element_type=_F32).astype(_BF)
        yt_ref[...] = jnp.dot(x0b, wt_ref[...].astype(_BF),
                              preferred_element_type=_F32).astype(_BF)
        y1_ref[...] = jnp.dot(x1b, w1_ref[...].astype(_BF),
                              preferred_element_type=_F32).astype(_BF)
        ys_ref[...] = jnp.dot(x1b, ws_ref[...].astype(_BF),
                              preferred_element_type=_F32).astype(_BF)

    a0 = a0_ref[...].astype(_BF)
    a1 = a1_ref[...].astype(_BF)
    b = b_ref[...].astype(_BF)

    m0 = jnp.dot(a0, y0_ref[pl.ds(j * A0K, A0K), :],
                 preferred_element_type=_F32)
    m0 += jnp.dot(b, ys_ref[pl.ds(j * BJ, BJ), :],
                  preferred_element_type=_F32)

    m1 = jnp.dot(a1, y1_ref[pl.ds(i * A1K, A1K), :],
                 preferred_element_type=_F32)
    m1 += jax.lax.dot_general(b, yt_ref[pl.ds(i * BI, BI), :],
                              dimension_numbers=(((0,), (0,)), ((), ())),
                              preferred_element_type=_F32)

    @pl.when(j == 0)
    def _():
        out0_ref[pl.ds(i * BI, BI), :] = m0

    @pl.when(j > 0)
    def _():
        out0_ref[pl.ds(i * BI, BI), :] += m0

    @pl.when(i == 0)
    def _():
        out1_ref[pl.ds(j * BJ, BJ), :] = m1

    @pl.when(i > 0)
    def _():
        out1_ref[pl.ds(j * BJ, BJ), :] += m1


def kernel(x_0, x_1, x_2, x_3, x_4, adjacency_0, adjacency_1, adjacency_2,
           adjacency_3, adjacency_4, incidence_0_1, incidence_0_2,
           incidence_0_3, incidence_0_4, incidence_1_2, incidence_1_3,
           incidence_1_4, incidence_2_3, incidence_2_4, incidence_3_4,
           w_hbs0, w_hbs1, w_hbns_s, w_hbns_t):
    grid = (GI, GJ)
    full = lambda shape: pl.BlockSpec(shape, lambda i, j: (0, 0))
    out0, out1 = pl.pallas_call(
        _gnn_kernel,
        grid=grid,
        in_specs=[
            full((N0, D)),
            full((N1, D)),
            pl.BlockSpec((BI, A0K), lambda i, j: (i, j)),
            pl.BlockSpec((BJ, A1K), lambda i, j: (j, i)),
            pl.BlockSpec((BI, BJ), lambda i, j: (i, j)),
            full((D, D)), full((D, D)), full((D, D)), full((D, D)),
        ],
        out_specs=[full((N0, D)), full((N1, D))],
        out_shape=[
            jax.ShapeDtypeStruct((N0, D), _F32),
            jax.ShapeDtypeStruct((N1, D), _F32),
        ],
        scratch_shapes=[
            pltpu.VMEM((N0, D), _BF),
            pltpu.VMEM((N1, D), _BF),
            pltpu.VMEM((N1, D), _BF),
            pltpu.VMEM((N0, D), _BF),
        ],
    )(x_0, x_1, adjacency_0, adjacency_1, incidence_0_1,
      w_hbs0, w_hbs1, w_hbns_s, w_hbns_t)
    return (out0, out1, x_2, x_3, x_4)
